# Initial kernel scaffold; baseline (speedup 1.0000x reference)
import jax
import jax.numpy as jnp
from jax.experimental import pallas as pl


def kernel(x, edge_index, edge_weight, edge_attr, W_lin1, W_nn1, b_nn1, W_nn2, b_nn2, W_lin2, b_lin2, W_out, b_out):
    raise NotImplementedError("write your pallas kernel here")


# speedup vs baseline: 1.4929x; 1.4929x over previous
import functools
import math

import jax
import jax.numpy as jnp
from jax import lax
from jax.experimental import pallas as pl
from jax.experimental.pallas import tpu as pltpu
from jax.experimental.pallas import tpu_sc as plsc

_LOG2 = math.log(2.0)

_NC = 2
_NS = 16
_NW = _NC * _NS


def _filter_body(ea_ref, ew_ref, w1_ref, b1_ref, w2_ref, b2_ref, out_ref, *, cutoff):
    z = jnp.dot(ea_ref[...], w1_ref[...], preferred_element_type=jnp.float32)
    z = z + b1_ref[...]
    z = jnp.maximum(z, 0.0) + jnp.log(1.0 + jnp.exp(-jnp.abs(z))) - _LOG2
    w = jnp.dot(z, w2_ref[...], preferred_element_type=jnp.float32) + b2_ref[...]
    c = 0.5 * (jnp.cos(ew_ref[...] * (math.pi / cutoff)) + 1.0)
    out_ref[...] = w * c


def _edge_filters(edge_attr, edge_weight, W_nn1, b_nn1, W_nn2, b_nn2, cutoff):
    E, R = edge_attr.shape
    H = W_nn1.shape[1]
    EB = 4000
    grid = (E // EB,)
    return pl.pallas_call(
        functools.partial(_filter_body, cutoff=cutoff),
        grid=grid,
        in_specs=[
            pl.BlockSpec((EB, R), lambda i: (i, 0)),
            pl.BlockSpec((EB, 1), lambda i: (i, 0)),
            pl.BlockSpec((R, H), lambda i: (0, 0)),
            pl.BlockSpec((1, H), lambda i: (0, 0)),
            pl.BlockSpec((H, H), lambda i: (0, 0)),
            pl.BlockSpec((1, H), lambda i: (0, 0)),
        ],
        out_specs=pl.BlockSpec((EB, H), lambda i: (i, 0)),
        out_shape=jax.ShapeDtypeStruct((E, H), jnp.float32),
    )(edge_attr, edge_weight.reshape(E, 1), W_nn1, b_nn1.reshape(1, H),
      W_nn2, b_nn2.reshape(1, H))


def _lin1_body(x_ref, w_ref, out_ref):
    out_ref[...] = jnp.dot(x_ref[...], w_ref[...], preferred_element_type=jnp.float32)


def _lin1(x, W_lin1):
    N, H = x.shape
    return pl.pallas_call(
        _lin1_body,
        out_shape=jax.ShapeDtypeStruct((N, H), jnp.float32),
    )(x, W_lin1)


def _make_scatter(NPAD, E, H):
    EPT = E // _NW
    CH = 80
    n_chunks = EPT // CH
    RPT = NPAD // _NS

    mesh = plsc.VectorSubcoreMesh(core_axis_name="c", subcore_axis_name="s")

    @functools.partial(
        pl.kernel,
        mesh=mesh,
        out_type=pltpu.HBM((_NC, NPAD, H), jnp.float32),
        scratch_types=[
            pltpu.VMEM((CH,), jnp.int32),
            pltpu.VMEM((CH,), jnp.int32),
            pltpu.VMEM((CH, H), jnp.float32),
            pltpu.VMEM((CH, H), jnp.float32),
            pltpu.VMEM_SHARED((NPAD, H), jnp.float32),
            pltpu.SemaphoreType.DMA,
        ],
    )
    def sc_kernel(h_hbm, wf_hbm, src_hbm, dst_hbm, out_hbm,
                  src_v, dst_v, wf_v, rows_v, agg_sh, sem):
        c = lax.axis_index("c")
        s = lax.axis_index("s")
        wid = s * _NC + c

        def zrow(r, _):
            for cb in range(H // 16):
                rows_v[r, pl.ds(cb * 16, 16)] = jnp.zeros((16,), jnp.float32)
            return 0
        lax.fori_loop(0, CH, zrow, 0)

        def zcopy(k, _):
            pltpu.sync_copy(rows_v, agg_sh.at[pl.ds(s * RPT + k * CH, CH), :])
            return 0
        lax.fori_loop(0, RPT // CH, zcopy, 0)
        plsc.subcore_barrier()

        def chunk(j, _):
            base = wid * EPT + j * CH
            pltpu.sync_copy(src_hbm.at[pl.ds(base, CH)], src_v)
            pltpu.sync_copy(dst_hbm.at[pl.ds(base, CH)], dst_v)
            pltpu.sync_copy(wf_hbm.at[pl.ds(base, CH), :], wf_v)
            pltpu.async_copy(h_hbm.at[src_v], rows_v, sem).wait()

            def mrow(r, _):
                for cb in range(H // 16):
                    sl = pl.ds(cb * 16, 16)
                    rows_v[r, sl] = rows_v[r, sl] * wf_v[r, sl]
                return 0
            lax.fori_loop(0, CH, mrow, 0)

            pltpu.sync_copy(rows_v, agg_sh.at[dst_v], add=True)
            return 0
        lax.fori_loop(0, n_chunks, chunk, 0)
        plsc.subcore_barrier()

        pltpu.sync_copy(agg_sh.at[pl.ds(s * RPT, RPT), :],
                        out_hbm.at[c, pl.ds(s * RPT, RPT), :])

    return sc_kernel


def _final_body(a0_ref, a1_ref, wl2_ref, bl2_ref, wo_ref, bo_ref, out_ref):
    a = a0_ref[...] + a1_ref[...]
    hh = jnp.dot(a, wl2_ref[...], preferred_element_type=jnp.float32) + bl2_ref[...]
    hh = jnp.tanh(hh)
    out_ref[...] = jnp.dot(hh, wo_ref[...], preferred_element_type=jnp.float32) + bo_ref[...]


def _final(a0, a1, W_lin2, b_lin2, W_out, b_out):
    N, H = a0.shape
    return pl.pallas_call(
        _final_body,
        out_shape=jax.ShapeDtypeStruct((N, H), jnp.float32),
    )(a0, a1, W_lin2, b_lin2.reshape(1, H), W_out, b_out.reshape(1, H))


def kernel(x, edge_index, edge_weight, edge_attr, W_lin1, W_nn1, b_nn1,
           W_nn2, b_nn2, W_lin2, b_lin2, W_out, b_out):
    N, H = x.shape
    E = edge_weight.shape[0]
    src = edge_index[0].astype(jnp.int32)
    dst = edge_index[1].astype(jnp.int32)

    h = _lin1(x, W_lin1)
    wf = _edge_filters(edge_attr, edge_weight, W_nn1, b_nn1, W_nn2, b_nn2, 10.0)
    npad = ((N + 80 * _NS - 1) // (80 * _NS)) * (80 * _NS)
    agg = _make_scatter(npad, E, H)(h, wf, src, dst)
    return _final(agg[0, :N], agg[1, :N], W_lin2, b_lin2, W_out, b_out)
